# Initial kernel scaffold; baseline (speedup 1.0000x reference)
import jax
import jax.numpy as jnp
from jax.experimental import pallas as pl


def kernel(x, W1, b1, W2, b2, ln1_g, ln1_b, Wpi, bpi, Wpo, bpo, W3, b3, W4, b4, ln2_g, ln2_b):
    raise NotImplementedError("write your pallas kernel here")


# speedup vs baseline: 1.2277x; 1.2277x over previous
import numpy as np
import jax
import jax.numpy as jnp
from jax.experimental import pallas as pl

_LEVELS = np.array([8, 8, 8, 5, 5, 5], dtype=np.int32)
_CDIM = 6
_CPAD = 128
_EPS_BOUND = 1e-3
_EPS_LN = 1e-5

_LEV_PAD = np.full((_CPAD,), 8, dtype=np.float32)
_LEV_PAD[:_CDIM] = _LEVELS.astype(np.float32)
_HALF_L = (_LEV_PAD - 1.0) * (1.0 + _EPS_BOUND) / 2.0
_OFFSET = np.where(_LEV_PAD.astype(np.int32) % 2 == 0, 0.5, 0.0).astype(np.float32)
_SHIFT = np.arctanh(_OFFSET / _HALF_L).astype(np.float32)
_INV_HALF = (1.0 / (_LEV_PAD.astype(np.int32) // 2).astype(np.float32)).astype(np.float32)


def _pipeline_kernel(x_ref, W1_ref, b1_ref, W2_ref, b2_ref, g1_ref, bb1_ref,
                     Wpi_ref, bpi_ref, Wpo_ref, bpo_ref, W3_ref, b3_ref,
                     W4_ref, b4_ref, g2_ref, bb2_ref, fsq_ref, out_ref):
    f32 = jnp.float32
    xb = x_ref[0]

    h = jnp.dot(xb, W1_ref[...], preferred_element_type=f32) + b1_ref[...]
    h = jnp.maximum(h, 0.0)
    h = jnp.dot(h, W2_ref[...], preferred_element_type=f32) + b2_ref[...]
    h = jnp.maximum(h, 0.0)

    mu = jnp.mean(h)
    var = jnp.mean((h - mu) ** 2)
    h = (h - mu) * jax.lax.rsqrt(var + _EPS_LN) * g1_ref[...] + bb1_ref[...]

    z = jnp.dot(h, Wpi_ref[...], preferred_element_type=f32) + bpi_ref[...]
    half_l = fsq_ref[0:1, :]
    offset = fsq_ref[1:2, :]
    inv_half = fsq_ref[2:3, :]
    bounded = jnp.tanh(z) * half_l - offset
    codes = jnp.round(bounded) * inv_half

    xq = jnp.dot(codes, Wpo_ref[...], preferred_element_type=f32) + bpo_ref[...]

    d = jnp.dot(xq, W3_ref[...], preferred_element_type=f32) + b3_ref[...]
    d = jnp.maximum(d, 0.0)
    d = jnp.dot(d, W4_ref[...], preferred_element_type=f32) + b4_ref[...]
    d = jnp.maximum(d, 0.0)

    mu2 = jnp.mean(d)
    var2 = jnp.mean((d - mu2) ** 2)
    out_ref[0] = (d - mu2) * jax.lax.rsqrt(var2 + _EPS_LN) * g2_ref[...] + bb2_ref[...]


def kernel(x, W1, b1, W2, b2, ln1_g, ln1_b, Wpi, bpi, Wpo, bpo, W3, b3, W4, b4, ln2_g, ln2_b):
    B, T, DIN = x.shape
    DE = W1.shape[1]
    DOUT = W4.shape[1]

    Wpi_pad = jnp.zeros((DE, _CPAD), jnp.float32).at[:, :_CDIM].set(Wpi)
    bpi_pad = (jnp.zeros((1, _CPAD), jnp.float32).at[0, :_CDIM].set(bpi)
               + jnp.asarray(_SHIFT)[None, :])
    Wpo_pad = jnp.zeros((_CPAD, DE), jnp.float32).at[:_CDIM, :].set(Wpo)
    fsq_const = jnp.asarray(
        np.stack([_HALF_L, _OFFSET, _INV_HALF] + [np.zeros_like(_HALF_L)] * 5))

    full = lambda shape: pl.BlockSpec(shape, lambda b: (0,) * len(shape))
    grid_spec = pl.GridSpec(
        grid=(B,),
        in_specs=[
            pl.BlockSpec((1, T, DIN), lambda b: (b, 0, 0)),
            full((DIN, DE)), full((1, DE)),
            full((DE, DE)), full((1, DE)),
            full((T, DE)), full((T, DE)),
            full((DE, _CPAD)), full((1, _CPAD)),
            full((_CPAD, DE)), full((1, DE)),
            full((DE, DE)), full((1, DE)),
            full((DE, DOUT)), full((1, DOUT)),
            full((T, DOUT)), full((T, DOUT)),
            full((8, _CPAD)),
        ],
        out_specs=pl.BlockSpec((1, T, DOUT), lambda b: (b, 0, 0)),
    )

    return pl.pallas_call(
        _pipeline_kernel,
        grid_spec=grid_spec,
        out_shape=jax.ShapeDtypeStruct((B, T, DOUT), jnp.float32),
    )(x, W1, b1.reshape(1, DE), W2, b2.reshape(1, DE), ln1_g, ln1_b,
      Wpi_pad, bpi_pad, Wpo_pad, bpo.reshape(1, DE), W3, b3.reshape(1, DE),
      W4, b4.reshape(1, DOUT), ln2_g, ln2_b, fsq_const)
